# baseline (speedup 1.0000x reference)
import functools

import jax
import jax.numpy as jnp
from jax import lax
from jax.experimental import pallas as pl
from jax.experimental.pallas import tpu as pltpu


def _k1_body(S, xyz1_ref, xyz2t_ref, p2t_ref, p1t_ref, w1at_ref, w1bt_ref,
             b1_ref, x1_ref, st_ref):
    xq = xyz1_ref[0]
    x2t = xyz2t_ref[0]
    blk = xq.shape[0]
    mm = jnp.dot(xq.astype(jnp.bfloat16), x2t.astype(jnp.bfloat16),
                 preferred_element_type=jnp.float32)
    s1 = xq[:, 0:1] * xq[:, 0:1]
    s1 = s1 + xq[:, 1:2] * xq[:, 1:2]
    s1 = s1 + xq[:, 2:3] * xq[:, 2:3]
    s2 = x2t[0:1, :] * x2t[0:1, :]
    s2 = s2 + x2t[1:2, :] * x2t[1:2, :]
    s2 = s2 + x2t[2:3, :] * x2t[2:3, :]
    d = -2.0 * mm
    d = d + s1
    d = d + s2
    d = jnp.maximum(d, 0.0)
    iota = lax.broadcasted_iota(jnp.int32, (blk, S), 1)
    wmat = jnp.zeros((blk, S), jnp.float32)
    norm = jnp.zeros((blk, 1), jnp.float32)
    dcur = d
    for r in range(3):
        m = jnp.min(dcur, axis=1, keepdims=True)
        is_min = dcur <= m
        idxk = jnp.min(jnp.where(is_min, iota, S), axis=1, keepdims=True)
        oh = iota == idxk
        wk = 1.0 / (m + 1e-8)
        norm = norm + wk
        wmat = jnp.where(oh, jnp.broadcast_to(wk, (blk, S)), wmat)
        if r < 2:
            dcur = jnp.where(oh, 1e30, dcur)
    interp = jnp.dot(wmat.astype(jnp.bfloat16), p2t_ref[0].astype(jnp.bfloat16),
                     preferred_element_type=jnp.float32)
    interp = interp / norm
    x1 = jnp.dot(p1t_ref[0].astype(jnp.bfloat16), w1at_ref[...].astype(jnp.bfloat16),
                 preferred_element_type=jnp.float32)
    x1 = x1 + jnp.dot(interp.astype(jnp.bfloat16), w1bt_ref[...].astype(jnp.bfloat16),
                      preferred_element_type=jnp.float32)
    x1 = x1 + b1_ref[0][None, :]
    x1_ref[0] = x1
    st_ref[0] = jnp.concatenate(
        [jnp.sum(x1, axis=0)[None], jnp.sum(x1 * x1, axis=0)[None]], axis=0)


def _k2_body(n, x1_ref, st1_ref, g1_ref, bt1_ref, w2t_ref, b2_ref,
             x2_ref, st2_ref):
    s = jnp.sum(st1_ref[...], axis=0)
    mean = s[0] / n
    var = s[1] / n - mean * mean
    a = g1_ref[0] * lax.rsqrt(var + 1e-5)
    c = bt1_ref[0] - mean * a
    y = jnp.maximum(x1_ref[0] * a[None, :] + c[None, :], 0.0)
    x2 = jnp.dot(y.astype(jnp.bfloat16), w2t_ref[...].astype(jnp.bfloat16),
                 preferred_element_type=jnp.float32)
    x2 = x2 + b2_ref[0][None, :]
    x2_ref[0] = x2
    st2_ref[0] = jnp.concatenate(
        [jnp.sum(x2, axis=0)[None], jnp.sum(x2 * x2, axis=0)[None]], axis=0)


def _k3_body(n, x2_ref, st2_ref, g2_ref, bt2_ref, out_ref):
    s = jnp.sum(st2_ref[...], axis=0)
    mean = s[0] / n
    var = s[1] / n - mean * mean
    a = g2_ref[0] * lax.rsqrt(var + 1e-5)
    c = bt2_ref[0] - mean * a
    out_ref[0] = jnp.maximum(x2_ref[0] * a[None, :] + c[None, :], 0.0)


@jax.jit
def kernel(xyz1, xyz2, points1, points2, conv1_W, conv1_b, bn1_gamma,
           bn1_beta, conv2_W, conv2_b, bn2_gamma, bn2_beta):
    B, N, _ = xyz1.shape
    S = xyz2.shape[1]
    C1 = points1.shape[1]
    C2 = points2.shape[1]
    CM = conv1_W.shape[0]
    CO = conv2_W.shape[0]
    n = float(B * N)

    blk1 = 512 if N % 512 == 0 else N
    blk2 = 2048 if N % 2048 == 0 else N
    nblk1 = N // blk1
    nblk2 = N // blk2

    xyz2t = jnp.swapaxes(xyz2, 1, 2)
    p1t = jnp.swapaxes(points1, 1, 2)
    p2t = jnp.swapaxes(points2, 1, 2)
    w1at = conv1_W[:, :C1].T
    w1bt = conv1_W[:, C1:].T
    w2t = conv2_W.T
    b1 = conv1_b[None, :]
    b2 = conv2_b[None, :]
    g1 = bn1_gamma[None, :]
    bt1 = bn1_beta[None, :]
    g2 = bn2_gamma[None, :]
    bt2 = bn2_beta[None, :]

    x1, st1 = pl.pallas_call(
        functools.partial(_k1_body, S),
        grid=(B, nblk1),
        in_specs=[
            pl.BlockSpec((1, blk1, 3), lambda b, i: (b, i, 0)),
            pl.BlockSpec((1, 3, S), lambda b, i: (b, 0, 0)),
            pl.BlockSpec((1, S, C2), lambda b, i: (b, 0, 0)),
            pl.BlockSpec((1, blk1, C1), lambda b, i: (b, i, 0)),
            pl.BlockSpec((C1, CM), lambda b, i: (0, 0)),
            pl.BlockSpec((C2, CM), lambda b, i: (0, 0)),
            pl.BlockSpec((1, CM), lambda b, i: (0, 0)),
        ],
        out_specs=[
            pl.BlockSpec((1, blk1, CM), lambda b, i: (b, i, 0)),
            pl.BlockSpec((1, 2, CM), lambda b, i: (b * nblk1 + i, 0, 0)),
        ],
        out_shape=[
            jax.ShapeDtypeStruct((B, N, CM), jnp.float32),
            jax.ShapeDtypeStruct((B * nblk1, 2, CM), jnp.float32),
        ],
        compiler_params=pltpu.CompilerParams(
            dimension_semantics=("parallel", "parallel")),
    )(xyz1, xyz2t, p2t, p1t, w1at, w1bt, b1)

    x2, st2 = pl.pallas_call(
        functools.partial(_k2_body, n),
        grid=(B, nblk2),
        in_specs=[
            pl.BlockSpec((1, blk2, CM), lambda b, i: (b, i, 0)),
            pl.BlockSpec((B * nblk1, 2, CM), lambda b, i: (0, 0, 0)),
            pl.BlockSpec((1, CM), lambda b, i: (0, 0)),
            pl.BlockSpec((1, CM), lambda b, i: (0, 0)),
            pl.BlockSpec((CM, CO), lambda b, i: (0, 0)),
            pl.BlockSpec((1, CO), lambda b, i: (0, 0)),
        ],
        out_specs=[
            pl.BlockSpec((1, blk2, CO), lambda b, i: (b, i, 0)),
            pl.BlockSpec((1, 2, CO), lambda b, i: (b * nblk2 + i, 0, 0)),
        ],
        out_shape=[
            jax.ShapeDtypeStruct((B, N, CO), jnp.float32),
            jax.ShapeDtypeStruct((B * nblk2, 2, CO), jnp.float32),
        ],
        compiler_params=pltpu.CompilerParams(
            dimension_semantics=("parallel", "parallel")),
    )(x1, st1, g1, bt1, w2t, b2)

    x3 = pl.pallas_call(
        functools.partial(_k3_body, n),
        grid=(B, nblk2),
        in_specs=[
            pl.BlockSpec((1, blk2, CO), lambda b, i: (b, i, 0)),
            pl.BlockSpec((B * nblk2, 2, CO), lambda b, i: (0, 0, 0)),
            pl.BlockSpec((1, CO), lambda b, i: (0, 0)),
            pl.BlockSpec((1, CO), lambda b, i: (0, 0)),
        ],
        out_specs=pl.BlockSpec((1, blk2, CO), lambda b, i: (b, i, 0)),
        out_shape=jax.ShapeDtypeStruct((B, N, CO), jnp.float32),
        compiler_params=pltpu.CompilerParams(
            dimension_semantics=("parallel", "parallel")),
    )(x2, st2, g2, bt2)

    return jnp.swapaxes(x3, 1, 2)


# speedup vs baseline: 1.1183x; 1.1183x over previous
import functools

import jax
import jax.numpy as jnp
from jax import lax
from jax.experimental import pallas as pl
from jax.experimental.pallas import tpu as pltpu


def _k1_body(S, xyz1_ref, xyz2t_ref, p2_ref, p1_ref, w1at_ref, w1bt_ref,
             b1_ref, x1_ref, st_ref):
    xq = xyz1_ref[0]
    x2t = xyz2t_ref[0]
    blk = xq.shape[0]
    mm = jnp.dot(xq.astype(jnp.bfloat16), x2t.astype(jnp.bfloat16),
                 preferred_element_type=jnp.float32)
    s1 = xq[:, 0:1] * xq[:, 0:1]
    s1 = s1 + xq[:, 1:2] * xq[:, 1:2]
    s1 = s1 + xq[:, 2:3] * xq[:, 2:3]
    s2 = x2t[0:1, :] * x2t[0:1, :]
    s2 = s2 + x2t[1:2, :] * x2t[1:2, :]
    s2 = s2 + x2t[2:3, :] * x2t[2:3, :]
    d = -2.0 * mm
    d = d + s1
    d = d + s2
    d = jnp.maximum(d, 0.0)
    iota = lax.broadcasted_iota(jnp.int32, (blk, S), 1)
    wmat = jnp.zeros((blk, S), jnp.float32)
    norm = jnp.zeros((blk, 1), jnp.float32)
    dcur = d
    for r in range(3):
        m = jnp.min(dcur, axis=1, keepdims=True)
        is_min = dcur <= m
        idxk = jnp.min(jnp.where(is_min, iota, S), axis=1, keepdims=True)
        oh = iota == idxk
        wk = 1.0 / (m + 1e-8)
        norm = norm + wk
        wmat = jnp.where(oh, jnp.broadcast_to(wk, (blk, S)), wmat)
        if r < 2:
            dcur = jnp.where(oh, 1e30, dcur)
    interp = lax.dot_general(
        wmat.astype(jnp.bfloat16), p2_ref[0].astype(jnp.bfloat16),
        (((1,), (1,)), ((), ())), preferred_element_type=jnp.float32)
    interp = interp / norm
    x1 = lax.dot_general(
        p1_ref[0].astype(jnp.bfloat16), w1at_ref[...].astype(jnp.bfloat16),
        (((0,), (0,)), ((), ())), preferred_element_type=jnp.float32)
    x1 = x1 + jnp.dot(interp.astype(jnp.bfloat16), w1bt_ref[...].astype(jnp.bfloat16),
                      preferred_element_type=jnp.float32)
    x1 = x1 + b1_ref[0][None, :]
    x1_ref[0] = x1
    st_ref[0] = jnp.concatenate(
        [jnp.sum(x1, axis=0)[None], jnp.sum(x1 * x1, axis=0)[None]], axis=0)


def _k2_body(n, x1_ref, st1_ref, g1_ref, bt1_ref, w2t_ref, b2_ref,
             x2_ref, st2_ref):
    s = jnp.sum(st1_ref[...], axis=0)
    mean = s[0] / n
    var = s[1] / n - mean * mean
    a = g1_ref[0] * lax.rsqrt(var + 1e-5)
    c = bt1_ref[0] - mean * a
    y = jnp.maximum(x1_ref[0] * a[None, :] + c[None, :], 0.0)
    x2 = jnp.dot(y.astype(jnp.bfloat16), w2t_ref[...].astype(jnp.bfloat16),
                 preferred_element_type=jnp.float32)
    x2 = x2 + b2_ref[0][None, :]
    x2_ref[0] = x2
    st2_ref[0] = jnp.concatenate(
        [jnp.sum(x2, axis=0)[None], jnp.sum(x2 * x2, axis=0)[None]], axis=0)


def _k3_body(n, x2_ref, st2_ref, g2_ref, bt2_ref, out_ref):
    s = jnp.sum(st2_ref[...], axis=0)
    mean = s[0] / n
    var = s[1] / n - mean * mean
    a = g2_ref[0] * lax.rsqrt(var + 1e-5)
    c = bt2_ref[0] - mean * a
    y = jnp.maximum(x2_ref[0] * a[None, :] + c[None, :], 0.0)
    out_ref[0] = y.T


@jax.jit
def kernel(xyz1, xyz2, points1, points2, conv1_W, conv1_b, bn1_gamma,
           bn1_beta, conv2_W, conv2_b, bn2_gamma, bn2_beta):
    B, N, _ = xyz1.shape
    S = xyz2.shape[1]
    C1 = points1.shape[1]
    C2 = points2.shape[1]
    CM = conv1_W.shape[0]
    CO = conv2_W.shape[0]
    n = float(B * N)

    blk1 = 512 if N % 512 == 0 else N
    blk2 = 2048 if N % 2048 == 0 else N
    nblk1 = N // blk1
    nblk2 = N // blk2

    xyz2t = jnp.swapaxes(xyz2, 1, 2)
    w1at = conv1_W[:, :C1].T
    w1bt = conv1_W[:, C1:].T
    w2t = conv2_W.T
    b1 = conv1_b[None, :]
    b2 = conv2_b[None, :]
    g1 = bn1_gamma[None, :]
    bt1 = bn1_beta[None, :]
    g2 = bn2_gamma[None, :]
    bt2 = bn2_beta[None, :]

    x1, st1 = pl.pallas_call(
        functools.partial(_k1_body, S),
        grid=(B, nblk1),
        in_specs=[
            pl.BlockSpec((1, blk1, 3), lambda b, i: (b, i, 0)),
            pl.BlockSpec((1, 3, S), lambda b, i: (b, 0, 0)),
            pl.BlockSpec((1, C2, S), lambda b, i: (b, 0, 0)),
            pl.BlockSpec((1, C1, blk1), lambda b, i: (b, 0, i)),
            pl.BlockSpec((C1, CM), lambda b, i: (0, 0)),
            pl.BlockSpec((C2, CM), lambda b, i: (0, 0)),
            pl.BlockSpec((1, CM), lambda b, i: (0, 0)),
        ],
        out_specs=[
            pl.BlockSpec((1, blk1, CM), lambda b, i: (b, i, 0)),
            pl.BlockSpec((1, 2, CM), lambda b, i: (b * nblk1 + i, 0, 0)),
        ],
        out_shape=[
            jax.ShapeDtypeStruct((B, N, CM), jnp.float32),
            jax.ShapeDtypeStruct((B * nblk1, 2, CM), jnp.float32),
        ],
        compiler_params=pltpu.CompilerParams(
            dimension_semantics=("parallel", "parallel")),
    )(xyz1, xyz2t, points2, points1, w1at, w1bt, b1)

    x2, st2 = pl.pallas_call(
        functools.partial(_k2_body, n),
        grid=(B, nblk2),
        in_specs=[
            pl.BlockSpec((1, blk2, CM), lambda b, i: (b, i, 0)),
            pl.BlockSpec((B * nblk1, 2, CM), lambda b, i: (0, 0, 0)),
            pl.BlockSpec((1, CM), lambda b, i: (0, 0)),
            pl.BlockSpec((1, CM), lambda b, i: (0, 0)),
            pl.BlockSpec((CM, CO), lambda b, i: (0, 0)),
            pl.BlockSpec((1, CO), lambda b, i: (0, 0)),
        ],
        out_specs=[
            pl.BlockSpec((1, blk2, CO), lambda b, i: (b, i, 0)),
            pl.BlockSpec((1, 2, CO), lambda b, i: (b * nblk2 + i, 0, 0)),
        ],
        out_shape=[
            jax.ShapeDtypeStruct((B, N, CO), jnp.float32),
            jax.ShapeDtypeStruct((B * nblk2, 2, CO), jnp.float32),
        ],
        compiler_params=pltpu.CompilerParams(
            dimension_semantics=("parallel", "parallel")),
    )(x1, st1, g1, bt1, w2t, b2)

    x3 = pl.pallas_call(
        functools.partial(_k3_body, n),
        grid=(B, nblk2),
        in_specs=[
            pl.BlockSpec((1, blk2, CO), lambda b, i: (b, i, 0)),
            pl.BlockSpec((B * nblk2, 2, CO), lambda b, i: (0, 0, 0)),
            pl.BlockSpec((1, CO), lambda b, i: (0, 0)),
            pl.BlockSpec((1, CO), lambda b, i: (0, 0)),
        ],
        out_specs=pl.BlockSpec((1, CO, blk2), lambda b, i: (b, 0, i)),
        out_shape=jax.ShapeDtypeStruct((B, CO, N), jnp.float32),
        compiler_params=pltpu.CompilerParams(
            dimension_semantics=("parallel", "parallel")),
    )(x2, st2, g2, bt2)

    return x3
